# baseline (speedup 1.0000x reference)
import functools

import jax
import jax.numpy as jnp
from jax import lax
from jax.experimental import pallas as pl
from jax.experimental.pallas import tpu as pltpu
from jax.experimental.pallas import tpu_sc as plsc

V = 1000000
D = 64
L = 20
B = 16384

_info = plsc.get_sparse_core_info()
NC, NS, LANES = _info.num_cores, _info.num_subcores, _info.num_lanes
NW = NC * NS
BPW = B // NW
GCHUNK = 128
NG = BPW // GCHUNK
NGRP = BPW // LANES


def _sc_body(idxT, table, w1h, w2h, out, idx_v, rows_v, acc_v, w1_v, w2_v, sem):
    wid = lax.axis_index("s") * NC + lax.axis_index("c")
    base = wid * BPW

    pltpu.sync_copy(w1h, w1_v)
    pltpu.sync_copy(w2h, w2_v)

    zeros = jnp.zeros((LANES,), jnp.float32)

    def zero_body(g, c):
        acc_v[pl.ds(g * LANES, LANES)] = zeros
        return c

    lax.fori_loop(0, NGRP, zero_body, 0)

    def l_body(l, c):
        pltpu.sync_copy(idxT.at[l, pl.ds(base, BPW)], idx_v)
        copies = [
            pltpu.async_copy(
                table.at[idx_v.at[pl.ds(k * GCHUNK, GCHUNK)]],
                rows_v.at[pl.ds(k * GCHUNK, GCHUNK), :],
                sem,
            )
            for k in range(NG)
        ]
        for cp in copies:
            cp.wait()
        wl = w2_v[pl.ds(l, LANES)][0]

        def g_body(g, cc):
            rbase = g * LANES
            row_idx = lax.iota(jnp.int32, LANES) + rbase
            w1segs = [w1_v[pl.ds(t * LANES, LANES)] for t in range(D // LANES)]
            s = jnp.zeros((LANES,), jnp.float32)
            for dd in range(D):
                col = jnp.full((LANES,), dd, jnp.int32)
                v = plsc.load_gather(rows_v, [row_idx, col])
                s = s + v * w1segs[dd // LANES][dd % LANES]
            idxs = idx_v[pl.ds(rbase, LANES)]
            contrib = jnp.where(idxs != 0, s, 0.0) * wl
            acc_v[pl.ds(rbase, LANES)] = acc_v[pl.ds(rbase, LANES)] + contrib
            return cc

        lax.fori_loop(0, NGRP, g_body, 0)
        return c

    lax.fori_loop(0, L, l_body, 0)

    const = w2_v[pl.ds(L, LANES)][0]

    def sig_body(g, c):
        x = acc_v[pl.ds(g * LANES, LANES)] + const
        acc_v[pl.ds(g * LANES, LANES)] = 1.0 / (1.0 + jnp.exp(-x))
        return c

    lax.fori_loop(0, NGRP, sig_body, 0)

    pltpu.sync_copy(acc_v, out.at[pl.ds(base, BPW)])


@functools.partial(
    pl.kernel,
    mesh=plsc.VectorSubcoreMesh(core_axis_name="c", subcore_axis_name="s"),
    out_type=jax.ShapeDtypeStruct((B,), jnp.float32),
    compiler_params=pltpu.CompilerParams(
        needs_layout_passes=False, use_tc_tiling_on_sc=False
    ),
    scratch_types=[
        pltpu.VMEM((BPW,), jnp.int32),
        pltpu.VMEM((BPW, D), jnp.float32),
        pltpu.VMEM((BPW,), jnp.float32),
        pltpu.VMEM((D,), jnp.float32),
        pltpu.VMEM((48,), jnp.float32),
        pltpu.SemaphoreType.DMA,
    ],
)
def _sc_kernel(idxT, table, w1h, w2h, out, idx_v, rows_v, acc_v, w1_v, w2_v, sem):
    _sc_body(idxT, table, w1h, w2h, out, idx_v, rows_v, acc_v, w1_v, w2_v, sem)


def kernel(char_inputs, table, w1, b1, w2, b2):
    idxT = jnp.transpose(char_inputs.astype(jnp.int32))
    w1f = w1[:, 0]
    const = b1[0] * jnp.sum(w2) + b2[0]
    w2p = jnp.zeros((48,), jnp.float32).at[:L].set(w2[:, 0]).at[L].set(const)
    out = _sc_kernel(idxT, table, w1f, w2p)
    return out.reshape(B, 1)


# speedup vs baseline: 1.1981x; 1.1981x over previous
import functools

import jax
import jax.numpy as jnp
from jax import lax
from jax.experimental import pallas as pl
from jax.experimental.pallas import tpu as pltpu
from jax.experimental.pallas import tpu_sc as plsc

V = 1000000
D = 64
L = 20
B = 16384

_info = plsc.get_sparse_core_info()
NC, NS, LANES = _info.num_cores, _info.num_subcores, _info.num_lanes
NW = NC * NS
BPW = B // NW
GCHUNK = 128
NG = BPW // GCHUNK
NGRP = BPW // LANES
NSEG = D // LANES


def _fire_gather(table, idx_all, l, buf, sem):
    for k in range(NG):
        pltpu.async_copy(
            table.at[idx_all.at[l, pl.ds(k * GCHUNK, GCHUNK)]],
            buf.at[pl.ds(k * GCHUNK, GCHUNK), :],
            sem,
        )


def _wait_gather(table, idx_all, buf, sem):
    pltpu.make_async_copy(table.at[idx_all.at[0, :]], buf, sem).wait()


def _compute_step(l, buf, idx_all, acc_v, stage_v, w1segs, w2_v, lane15, lane_iota):
    wl = w2_v[pl.ds(l, LANES)][0]

    def g_body(g, c):
        rbase = g * LANES
        for k in range(LANES):
            row = rbase + k
            s = buf[row, pl.ds(0, LANES)] * w1segs[0]
            for t in range(1, NSEG):
                s = s + buf[row, pl.ds(t * LANES, LANES)] * w1segs[t]
            cum = plsc.cumsum(s)
            plsc.store_scatter(
                stage_v, [jnp.full((LANES,), k, jnp.int32)], cum, mask=lane15
            )
        sdot = stage_v[...]
        idxs = idx_all[l, pl.ds(rbase, LANES)]
        contrib = jnp.where(idxs != 0, sdot, 0.0) * wl
        acc_v[pl.ds(rbase, LANES)] = acc_v[pl.ds(rbase, LANES)] + contrib
        return c

    lax.fori_loop(0, NGRP, g_body, 0)


def _sc_body(idxT, table, w1h, w2h, out, idx_all, rows0, rows1, acc_v, stage_v,
             w1_v, w2_v, sem0, sem1):
    wid = lax.axis_index("s") * NC + lax.axis_index("c")
    base = wid * BPW

    pltpu.sync_copy(w1h, w1_v)
    pltpu.sync_copy(w2h, w2_v)
    pltpu.sync_copy(idxT.at[:, pl.ds(base, BPW)], idx_all)

    zeros = jnp.zeros((LANES,), jnp.float32)

    def zero_body(g, c):
        acc_v[pl.ds(g * LANES, LANES)] = zeros
        return c

    lax.fori_loop(0, NGRP, zero_body, 0)

    lane_iota = lax.iota(jnp.int32, LANES)
    lane15 = lane_iota == (LANES - 1)

    _fire_gather(table, idx_all, 0, rows0, sem0)
    _fire_gather(table, idx_all, 1, rows1, sem1)

    def l_pair(i, c):
        l0 = 2 * i
        w1segs = [w1_v[pl.ds(t * LANES, LANES)] for t in range(NSEG)]

        _wait_gather(table, idx_all, rows0, sem0)
        _compute_step(l0, rows0, idx_all, acc_v, stage_v, w1segs, w2_v,
                      lane15, lane_iota)

        @pl.when(i < (L // 2 - 1))
        def _():
            _fire_gather(table, idx_all, l0 + 2, rows0, sem0)

        _wait_gather(table, idx_all, rows1, sem1)
        _compute_step(l0 + 1, rows1, idx_all, acc_v, stage_v, w1segs, w2_v,
                      lane15, lane_iota)

        @pl.when(i < (L // 2 - 1))
        def _():
            _fire_gather(table, idx_all, l0 + 3, rows1, sem1)

        return c

    lax.fori_loop(0, L // 2, l_pair, 0)

    const = w2_v[pl.ds(L, LANES)][0]

    def sig_body(g, c):
        x = acc_v[pl.ds(g * LANES, LANES)] + const
        acc_v[pl.ds(g * LANES, LANES)] = 1.0 / (1.0 + jnp.exp(-x))
        return c

    lax.fori_loop(0, NGRP, sig_body, 0)

    pltpu.sync_copy(acc_v, out.at[pl.ds(base, BPW)])


@functools.partial(
    pl.kernel,
    mesh=plsc.VectorSubcoreMesh(core_axis_name="c", subcore_axis_name="s"),
    out_type=jax.ShapeDtypeStruct((B,), jnp.float32),
    compiler_params=pltpu.CompilerParams(
        needs_layout_passes=False, use_tc_tiling_on_sc=False
    ),
    scratch_types=[
        pltpu.VMEM((L, BPW), jnp.int32),
        pltpu.VMEM((BPW, D), jnp.float32),
        pltpu.VMEM((BPW, D), jnp.float32),
        pltpu.VMEM((BPW,), jnp.float32),
        pltpu.VMEM((LANES,), jnp.float32),
        pltpu.VMEM((D,), jnp.float32),
        pltpu.VMEM((48,), jnp.float32),
        pltpu.SemaphoreType.DMA,
        pltpu.SemaphoreType.DMA,
    ],
)
def _sc_kernel(idxT, table, w1h, w2h, out, idx_all, rows0, rows1, acc_v,
               stage_v, w1_v, w2_v, sem0, sem1):
    _sc_body(idxT, table, w1h, w2h, out, idx_all, rows0, rows1, acc_v,
             stage_v, w1_v, w2_v, sem0, sem1)


def kernel(char_inputs, table, w1, b1, w2, b2):
    idxT = jnp.transpose(char_inputs.astype(jnp.int32))
    w1f = w1[:, 0]
    const = b1[0] * jnp.sum(w2) + b2[0]
    w2p = jnp.zeros((48,), jnp.float32).at[:L].set(w2[:, 0]).at[L].set(const)
    out = _sc_kernel(idxT, table, w1f, w2p)
    return out.reshape(B, 1)


# speedup vs baseline: 1.3219x; 1.1034x over previous
import functools

import jax
import jax.numpy as jnp
from jax import lax
from jax.experimental import pallas as pl
from jax.experimental.pallas import tpu as pltpu
from jax.experimental.pallas import tpu_sc as plsc

V = 1000000
D = 64
L = 20
B = 16384

_info = plsc.get_sparse_core_info()
NC, NS, LANES = _info.num_cores, _info.num_subcores, _info.num_lanes
NW = NC * NS
BPW = B // NW
GCHUNK = 128
NG = BPW // GCHUNK
NGRP = BPW // LANES
QBLK = 8
NBLK = NGRP // QBLK


def _fire_gather(table, idxT, l, buf, sem):
    for k in range(NG):
        pltpu.async_copy(
            table.at[idxT.at[l, pl.ds(k * GCHUNK, GCHUNK)]],
            buf.at[pl.ds(k * GCHUNK, GCHUNK), :],
            sem,
        )


def _wait_gather(table, idxT, buf, sem):
    pltpu.make_async_copy(table.at[idxT.at[0, :]], buf, sem).wait()


def _compute_step(l, buf, idxT, acc_v, w1rot_v, w2_v, lane_iota):
    wl = w2_v[pl.ds(l, LANES)][0]

    def blk_body(bk, c):
        rbase0 = bk * (QBLK * LANES)
        row_idx = [rbase0 + q * LANES + lane_iota for q in range(QBLK)]
        s = [jnp.zeros((LANES,), jnp.float32) for _ in range(QBLK)]
        colv = lane_iota
        for t in range(D):
            w = w1rot_v[t, :]
            for q in range(QBLK):
                v = plsc.load_gather(buf, [row_idx[q], colv])
                s[q] = s[q] + v * w
            colv = (colv + 1) & (D - 1)
        for q in range(QBLK):
            rb = rbase0 + q * LANES
            idxs = idxT[l, pl.ds(rb, LANES)]
            contrib = jnp.where(idxs != 0, s[q], 0.0) * wl
            acc_v[pl.ds(rb, LANES)] = acc_v[pl.ds(rb, LANES)] + contrib
        return c

    lax.fori_loop(0, NBLK, blk_body, 0)


def _sc_body(idx, table, w1rot_h, w2h, out, idx_loc, idxT, rows0, rows1,
             acc_v, w1rot_v, w2_v, sem0, sem1):
    wid = lax.axis_index("s") * NC + lax.axis_index("c")
    base = wid * BPW

    pltpu.sync_copy(idx.at[pl.ds(base, BPW), :], idx_loc)
    pltpu.sync_copy(w1rot_h, w1rot_v)
    pltpu.sync_copy(w2h, w2_v)

    lane_iota = lax.iota(jnp.int32, LANES)
    zeros = jnp.zeros((LANES,), jnp.float32)

    def tr_l(l, c):
        lvec = jnp.full((LANES,), l, jnp.int32)

        def tr_g(g, c2):
            rows = g * LANES + lane_iota
            v = plsc.load_gather(idx_loc, [rows, lvec])
            idxT[l, pl.ds(g * LANES, LANES)] = v
            return c2

        lax.fori_loop(0, NGRP, tr_g, 0)
        return c

    lax.fori_loop(0, L, tr_l, 0)

    def zero_body(g, c):
        acc_v[pl.ds(g * LANES, LANES)] = zeros
        return c

    lax.fori_loop(0, NGRP, zero_body, 0)

    _fire_gather(table, idxT, 0, rows0, sem0)
    _fire_gather(table, idxT, 1, rows1, sem1)

    def l_pair(i, c):
        l0 = 2 * i

        _wait_gather(table, idxT, rows0, sem0)
        _compute_step(l0, rows0, idxT, acc_v, w1rot_v, w2_v, lane_iota)

        @pl.when(i < (L // 2 - 1))
        def _():
            _fire_gather(table, idxT, l0 + 2, rows0, sem0)

        _wait_gather(table, idxT, rows1, sem1)
        _compute_step(l0 + 1, rows1, idxT, acc_v, w1rot_v, w2_v, lane_iota)

        @pl.when(i < (L // 2 - 1))
        def _():
            _fire_gather(table, idxT, l0 + 3, rows1, sem1)

        return c

    lax.fori_loop(0, L // 2, l_pair, 0)

    const = w2_v[pl.ds(L, LANES)][0]

    def sig_body(g, c):
        x = acc_v[pl.ds(g * LANES, LANES)] + const
        acc_v[pl.ds(g * LANES, LANES)] = 1.0 / (1.0 + jnp.exp(-x))
        return c

    lax.fori_loop(0, NGRP, sig_body, 0)

    pltpu.sync_copy(acc_v, out.at[pl.ds(base, BPW)])


@functools.partial(
    pl.kernel,
    mesh=plsc.VectorSubcoreMesh(core_axis_name="c", subcore_axis_name="s"),
    out_type=jax.ShapeDtypeStruct((B,), jnp.float32),
    compiler_params=pltpu.CompilerParams(
        needs_layout_passes=False, use_tc_tiling_on_sc=False
    ),
    scratch_types=[
        pltpu.VMEM((BPW, L), jnp.int32),
        pltpu.VMEM((L, BPW), jnp.int32),
        pltpu.VMEM((BPW, D), jnp.float32),
        pltpu.VMEM((BPW, D), jnp.float32),
        pltpu.VMEM((BPW,), jnp.float32),
        pltpu.VMEM((D, LANES), jnp.float32),
        pltpu.VMEM((48,), jnp.float32),
        pltpu.SemaphoreType.DMA,
        pltpu.SemaphoreType.DMA,
    ],
)
def _sc_kernel(idx, table, w1rot_h, w2h, out, idx_loc, idxT, rows0, rows1,
               acc_v, w1rot_v, w2_v, sem0, sem1):
    _sc_body(idx, table, w1rot_h, w2h, out, idx_loc, idxT, rows0, rows1,
             acc_v, w1rot_v, w2_v, sem0, sem1)


def kernel(char_inputs, table, w1, b1, w2, b2):
    idx = char_inputs.astype(jnp.int32)
    w1f = w1[:, 0]
    rot = (jnp.arange(D)[:, None] + jnp.arange(LANES)[None, :]) % D
    w1rot = w1f[rot]
    const = b1[0] * jnp.sum(w2) + b2[0]
    w2p = jnp.zeros((48,), jnp.float32).at[:L].set(w2[:, 0]).at[L].set(const)
    out = _sc_kernel(idx, table, w1rot, w2p)
    return out.reshape(B, 1)
